# baseline (speedup 1.0000x reference)
import functools

import jax
import jax.numpy as jnp
from jax.experimental import pallas as pl
from jax.experimental.pallas import tpu as pltpu
from jax.experimental.pallas import tpu_sc as plsc


def _selector_sc(B, S, D, dtype):
    mesh = plsc.ScalarSubcoreMesh(axis_name="c", num_cores=1)

    @functools.partial(
        pl.kernel,
        mesh=mesh,
        out_type=jax.ShapeDtypeStruct((B, D), dtype),
        scratch_types=[
            pltpu.SMEM((B,), jnp.int32),
            pltpu.SemaphoreType.DMA,
        ],
    )
    def gather_kernel(x_hbm, idx_hbm, out_hbm, idx_s, sem):
        pltpu.sync_copy(idx_hbm, idx_s)
        copies = []
        for b in range(B):
            copies.append(
                pltpu.make_async_copy(
                    x_hbm.at[pl.ds(idx_s[b] + b * S, 1)],
                    out_hbm.at[pl.ds(b, 1)],
                    sem,
                )
            )
        for c in copies:
            c.start()
        for c in copies:
            c.wait()

    return gather_kernel


def kernel(x, idx):
    B, S, D = x.shape
    x_flat = x.reshape(B * S, D)
    return _selector_sc(B, S, D, x.dtype)(x_flat, idx.astype(jnp.int32))


# speedup vs baseline: 6.4143x; 6.4143x over previous
import jax
import jax.numpy as jnp
from jax.experimental import pallas as pl
from jax.experimental.pallas import tpu as pltpu


def kernel(x, idx):
    B, S, D = x.shape
    x_flat = x.reshape(B * S, D)

    def body(idx_ref, x_hbm, o_hbm, sem):
        copies = []
        for b in range(B):
            copies.append(
                pltpu.make_async_copy(
                    x_hbm.at[pl.ds(idx_ref[b] + b * S, 1)],
                    o_hbm.at[pl.ds(b, 1)],
                    sem,
                )
            )
        for c in copies:
            c.start()
        for c in copies:
            c.wait()

    return pl.pallas_call(
        body,
        in_specs=[
            pl.BlockSpec(memory_space=pltpu.SMEM),
            pl.BlockSpec(memory_space=pl.ANY),
        ],
        out_specs=pl.BlockSpec(memory_space=pl.ANY),
        out_shape=jax.ShapeDtypeStruct((B, D), x.dtype),
        scratch_shapes=[pltpu.SemaphoreType.DMA],
    )(idx.astype(jnp.int32), x_flat)
